# baseline (speedup 1.0000x reference)
import jax
import jax.numpy as jnp
from jax import lax
from jax.experimental import pallas as pl
from jax.experimental.pallas import tpu as pltpu
from jax.experimental.pallas import tpu_sc as plsc

_D = 2048
_B = 4
_SP = 8193
_SPAD = 8200
_WPB = 8
_ROWS_W = 1024
_C = 16
_NCH = _ROWS_W // _C
_NBUF = 2
_NC = 2
_IDSTAGE = 1032


def _sc_body(ids_hbm, zeros_hbm, out_hbm, ids_v, buf0, buf1, sem0, sem1):
    cid = lax.axis_index("c")
    sid = lax.axis_index("s")
    wid = sid * _NC + cid
    b = wid // _WPB
    sub = wid % _WPB
    row0 = sub * _ROWS_W

    pltpu.sync_copy(ids_hbm.at[pl.ds(b * _SPAD + row0, _IDSTAGE)],
                    ids_v.at[pl.ds(0, _IDSTAGE)])
    pltpu.sync_copy(zeros_hbm, buf0)
    pltpu.sync_copy(zeros_hbm, buf1)

    iota = lax.iota(jnp.int32, 16)
    ones_v = jnp.full((16,), 1.0, jnp.float32)
    zeros_v = jnp.zeros((16,), jnp.float32)
    bufs = (buf0, buf1)
    sems = (sem0, sem1)

    def _dst(j):
        return out_hbm.at[b, pl.ds(row0 + j * _C, _C), :]

    @pl.loop(0, _NCH, step=_NBUF)
    def _(g):
        for s in range(_NBUF):
            j = g + s

            @pl.when(j >= _NBUF)
            def _():
                pltpu.make_async_copy(bufs[s], _dst(j - _NBUF), sems[s]).wait()
                ids_old = ids_v[pl.ds((j - _NBUF) * _C, _C)]
                plsc.store_scatter(bufs[s], [iota, ids_old], zeros_v)

            ids_j = ids_v[pl.ds(j * _C, _C)]
            plsc.store_scatter(bufs[s], [iota, ids_j], ones_v)
            pltpu.async_copy(bufs[s], _dst(j), sems[s])

    for s in range(_NBUF):
        pltpu.make_async_copy(bufs[s], _dst(_NCH - _NBUF + s), sems[s]).wait()

    @pl.when(sub == _WPB - 1)
    def _():
        ids_old = ids_v[pl.ds((_NCH - _NBUF) * _C, _C)]
        plsc.store_scatter(buf0, [iota, ids_old], zeros_v)
        ids_tail = ids_v[pl.ds(_ROWS_W, _C)]
        lane0 = iota == 0
        plsc.store_scatter(buf0, [iota * 0, ids_tail], ones_v, mask=lane0)
        pltpu.async_copy(buf0.at[pl.ds(0, 1), :],
                         out_hbm.at[b, pl.ds(_WPB * _ROWS_W, 1), :], sem0)
        pltpu.make_async_copy(buf0.at[pl.ds(0, 1), :],
                              out_hbm.at[b, pl.ds(_WPB * _ROWS_W, 1), :],
                              sem0).wait()


_MESH = plsc.VectorSubcoreMesh(core_axis_name="c", subcore_axis_name="s")

_SC_CALL = pl.kernel(
    _sc_body,
    out_type=jax.ShapeDtypeStruct((_B, _SP, _D), jnp.float32),
    mesh=_MESH,
    scratch_types=[
        pltpu.VMEM((1040,), jnp.int32),
        pltpu.VMEM((_C, _D), jnp.float32),
        pltpu.VMEM((_C, _D), jnp.float32),
        pltpu.SemaphoreType.DMA,
        pltpu.SemaphoreType.DMA,
    ],
    compiler_params=pltpu.CompilerParams(needs_layout_passes=False),
)


def kernel(input_ids):
    b, s = input_ids.shape
    ids = input_ids.astype(jnp.int32)
    padded = jnp.concatenate([jnp.zeros((b, 1), jnp.int32), ids], axis=1)
    ids2d = jnp.pad(padded, ((0, 0), (0, _SPAD - _SP))).reshape(-1)
    zeros = jnp.zeros((_C, _D), jnp.float32)
    return _SC_CALL(ids2d, zeros)
